# baseline (speedup 1.0000x reference)
import jax
import jax.numpy as jnp
from jax import lax
from jax.experimental import pallas as pl
from jax.experimental.pallas import tpu as pltpu
from jax.experimental.pallas import tpu_sc as plsc

N = 10000
NP = 10240
E = 320000
C = 128
NEG = 0.2

NW = 32
CHUNK = 128
NCHUNK = 80
EPW = NCHUNK * CHUNK
EP = NW * EPW
RPT = NP // 16
BLK = 1024
GRID = NP // BLK


def _mm_body(x_ref, w_ref, asrc_ref, adst_ref, xp_ref, as_ref, ad_ref):
    xp = jnp.dot(x_ref[...], w_ref[...], preferred_element_type=jnp.float32)
    xp_ref[...] = xp
    as_ref[...] = jnp.sum(xp * asrc_ref[...], axis=1)
    ad_ref[...] = jnp.sum(xp * adst_ref[...], axis=1)


def _mm_call(xpad, W, att_src, att_dst):
    return pl.pallas_call(
        _mm_body,
        grid=(GRID,),
        in_specs=[
            pl.BlockSpec((BLK, C), lambda i: (i, 0)),
            pl.BlockSpec((C, C), lambda i: (0, 0)),
            pl.BlockSpec((1, C), lambda i: (0, 0)),
            pl.BlockSpec((1, C), lambda i: (0, 0)),
        ],
        out_specs=[
            pl.BlockSpec((BLK, C), lambda i: (i, 0)),
            pl.BlockSpec((BLK,), lambda i: (i,)),
            pl.BlockSpec((BLK,), lambda i: (i,)),
        ],
        out_shape=[
            jax.ShapeDtypeStruct((NP, C), jnp.float32),
            jax.ShapeDtypeStruct((NP,), jnp.float32),
            jax.ShapeDtypeStruct((NP,), jnp.float32),
        ],
    )(xpad, W, att_src, att_dst)


def _sc_body(xp_hbm, asrc_hbm, adst_hbm, src_hbm, dst_hbm,
             accp_hbm, denp_hbm,
             asrc_t, adst_t, sidx, didx, wbuf, rows, den_l, acc_sh):
    c = lax.axis_index("c")
    s = lax.axis_index("s")
    wid = s * 2 + c

    pltpu.sync_copy(asrc_hbm, asrc_t)
    pltpu.sync_copy(adst_hbm, adst_t)

    zero16 = jnp.zeros((16,), jnp.float32)

    def zden(i, _):
        den_l[pl.ds(i * 16, 16)] = zero16
        return 0
    lax.fori_loop(0, NP // 16, zden, 0)

    def zrow(i, _):
        for j in range(8):
            rows[i, pl.ds(j * 16, 16)] = zero16
        return 0
    lax.fori_loop(0, CHUNK, zrow, 0)

    for r in range(RPT // CHUNK):
        pltpu.sync_copy(rows, acc_sh.at[pl.ds(s * RPT + r * CHUNK, CHUNK), :])
    plsc.subcore_barrier()

    def rmax(tbl):
        def body(i, m):
            return jnp.maximum(m, tbl[pl.ds(i * 16, 16)])
        m16 = lax.fori_loop(0, NP // 16, body,
                            jnp.full((16,), -jnp.inf, jnp.float32))
        m = m16[0]
        for i in range(1, 16):
            m = jnp.maximum(m, m16[i])
        return m
    M = rmax(asrc_t) + rmax(adst_t)

    ebase = wid * EPW

    def chunk_body(cidx, _):
        off = ebase + cidx * CHUNK
        pltpu.sync_copy(src_hbm.at[pl.ds(off, CHUNK)], sidx)
        pltpu.sync_copy(dst_hbm.at[pl.ds(off, CHUNK)], didx)
        for j in range(CHUNK // 16):
            si = sidx[pl.ds(j * 16, 16)]
            di = didx[pl.ds(j * 16, 16)]
            e = plsc.load_gather(asrc_t, [si]) + plsc.load_gather(adst_t, [di])
            e = jnp.where(e > 0, e, NEG * e)
            w = jnp.exp(e - M)
            wbuf[pl.ds(j * 16, 16)] = w
            plsc.addupdate_scatter(den_l, [di], w)
        pltpu.sync_copy(xp_hbm.at[sidx], rows)

        def sgroup(g, _):
            wv = wbuf[pl.ds(g * 16, 16)]
            for rr in range(16):
                wr = wv[rr]
                r = g * 16 + rr
                for j in range(8):
                    rows[r, pl.ds(j * 16, 16)] = rows[r, pl.ds(j * 16, 16)] * wr
            return 0
        lax.fori_loop(0, CHUNK // 16, sgroup, 0)
        pltpu.sync_copy(rows, acc_sh.at[didx], add=True)
        return 0
    lax.fori_loop(0, NCHUNK, chunk_body, 0)

    plsc.subcore_barrier()
    pltpu.sync_copy(acc_sh.at[pl.ds(s * RPT, RPT), :],
                    accp_hbm.at[c, pl.ds(s * RPT, RPT), :])
    pltpu.sync_copy(den_l, denp_hbm.at[wid])


def _sc_call(xp, asrc, adst, srcp, dstp):
    f = pl.kernel(
        _sc_body,
        out_type=(jax.ShapeDtypeStruct((2, NP, C), jnp.float32),
                  jax.ShapeDtypeStruct((NW, NP), jnp.float32)),
        mesh=plsc.VectorSubcoreMesh(core_axis_name="c", subcore_axis_name="s"),
        compiler_params=pltpu.CompilerParams(needs_layout_passes=False),
        scratch_types=[
            pltpu.VMEM((NP,), jnp.float32),
            pltpu.VMEM((NP,), jnp.float32),
            pltpu.VMEM((CHUNK,), jnp.int32),
            pltpu.VMEM((CHUNK,), jnp.int32),
            pltpu.VMEM((CHUNK,), jnp.float32),
            pltpu.VMEM((CHUNK, C), jnp.float32),
            pltpu.VMEM((NP,), jnp.float32),
            pltpu.VMEM_SHARED((NP, C), jnp.float32),
        ],
    )
    return f(xp, asrc, adst, srcp, dstp)


def _comb_body(acc_ref, den_ref, bias_ref, out_ref):
    a = acc_ref[0] + acc_ref[1]
    d = jnp.sum(den_ref[...], axis=0)
    out_ref[...] = a / (d + 1e-16)[:, None] + bias_ref[...]


def _comb_call(accp, denp, bias):
    return pl.pallas_call(
        _comb_body,
        grid=(GRID,),
        in_specs=[
            pl.BlockSpec((2, BLK, C), lambda i: (0, i, 0)),
            pl.BlockSpec((NW, BLK), lambda i: (0, i)),
            pl.BlockSpec((1, C), lambda i: (0, 0)),
        ],
        out_specs=pl.BlockSpec((BLK, C), lambda i: (i, 0)),
        out_shape=jax.ShapeDtypeStruct((NP, C), jnp.float32),
    )(accp, denp, bias)


def kernel(x, edge_index, W, att_src, att_dst, bias):
    xpad = jnp.pad(x, ((0, NP - N), (0, 0)))
    srcp = jnp.pad(edge_index[0], (0, EP - E), constant_values=NP - 1)
    dstp = jnp.pad(edge_index[1], (0, EP - E), constant_values=NP - 1)
    xp, asrc, adst = _mm_call(xpad, W, att_src.reshape(1, C),
                              att_dst.reshape(1, C))
    accp, denp = _sc_call(xp, asrc, adst, srcp, dstp)
    out = _comb_call(accp, denp, bias.reshape(1, C))
    return out[:N]


# speedup vs baseline: 1.2757x; 1.2757x over previous
import jax
import jax.numpy as jnp
from jax import lax
from jax.experimental import pallas as pl
from jax.experimental.pallas import tpu as pltpu
from jax.experimental.pallas import tpu_sc as plsc

N = 10000
NP = 10240
E = 320000
C = 128
NEG = 0.2

NW = 32
CHUNK = 128
NCHUNK = 79
EPW = NCHUNK * CHUNK
EP = NW * EPW
RPT = NP // 16
BLK = 1024
GRID = NP // BLK


def _mm_body(x_ref, w_ref, asrc_ref, adst_ref, xp_ref, as_ref, ad_ref):
    xp = jnp.dot(x_ref[...], w_ref[...], preferred_element_type=jnp.float32)
    xp_ref[...] = xp
    as_ref[...] = jnp.sum(xp * asrc_ref[...], axis=1)
    ad_ref[...] = jnp.sum(xp * adst_ref[...], axis=1)


def _mm_call(xpad, W, att_src, att_dst):
    return pl.pallas_call(
        _mm_body,
        grid=(GRID,),
        in_specs=[
            pl.BlockSpec((BLK, C), lambda i: (i, 0)),
            pl.BlockSpec((C, C), lambda i: (0, 0)),
            pl.BlockSpec((1, C), lambda i: (0, 0)),
            pl.BlockSpec((1, C), lambda i: (0, 0)),
        ],
        out_specs=[
            pl.BlockSpec((BLK, C), lambda i: (i, 0)),
            pl.BlockSpec((BLK,), lambda i: (i,)),
            pl.BlockSpec((BLK,), lambda i: (i,)),
        ],
        out_shape=[
            jax.ShapeDtypeStruct((NP, C), jnp.float32),
            jax.ShapeDtypeStruct((NP,), jnp.float32),
            jax.ShapeDtypeStruct((NP,), jnp.float32),
        ],
    )(xpad, W, att_src, att_dst)


def _sc_body(xp_hbm, asrc_hbm, adst_hbm, src_hbm, dst_hbm,
             accp_hbm, denp_hbm,
             asrc_t, adst_t, sidx, didx, wbuf, rows, den_l, acc_sh):
    c = lax.axis_index("c")
    s = lax.axis_index("s")
    wid = s * 2 + c

    pltpu.sync_copy(asrc_hbm, asrc_t)
    pltpu.sync_copy(adst_hbm, adst_t)

    zero16 = jnp.zeros((16,), jnp.float32)

    def zden(i, _):
        den_l[pl.ds(i * 16, 16)] = zero16
        return 0
    lax.fori_loop(0, NP // 16, zden, 0)

    def zrow(i, _):
        for j in range(8):
            rows[i, pl.ds(j * 16, 16)] = zero16
        return 0
    lax.fori_loop(0, CHUNK, zrow, 0)

    for r in range(RPT // CHUNK):
        pltpu.sync_copy(rows, acc_sh.at[pl.ds(s * RPT + r * CHUNK, CHUNK), :])
    plsc.subcore_barrier()

    def rmax(tbl):
        def body(i, m):
            return jnp.maximum(m, tbl[pl.ds(i * 16, 16)])
        m16 = lax.fori_loop(0, NP // 16, body,
                            jnp.full((16,), -jnp.inf, jnp.float32))
        m = m16[0]
        for i in range(1, 16):
            m = jnp.maximum(m, m16[i])
        return m
    M = rmax(asrc_t) + rmax(adst_t)

    ebase = wid * EPW

    def chunk_body(cidx, _):
        off = ebase + cidx * CHUNK
        pltpu.sync_copy(src_hbm.at[pl.ds(off, CHUNK)], sidx)
        pltpu.sync_copy(dst_hbm.at[pl.ds(off, CHUNK)], didx)
        for j in range(CHUNK // 16):
            si = sidx[pl.ds(j * 16, 16)]
            di = didx[pl.ds(j * 16, 16)]
            e = plsc.load_gather(asrc_t, [si]) + plsc.load_gather(adst_t, [di])
            e = jnp.where(e > 0, e, NEG * e)
            w = jnp.exp(e - M)
            wbuf[pl.ds(j * 16, 16)] = w
            plsc.addupdate_scatter(den_l, [di], w)
        pltpu.sync_copy(xp_hbm.at[sidx], rows)

        def sgroup(g, _):
            wv = wbuf[pl.ds(g * 16, 16)]
            for rr in range(16):
                wr = wv[rr]
                r = g * 16 + rr
                for j in range(8):
                    rows[r, pl.ds(j * 16, 16)] = rows[r, pl.ds(j * 16, 16)] * wr
            return 0
        lax.fori_loop(0, CHUNK // 16, sgroup, 0)
        pltpu.sync_copy(rows, acc_sh.at[didx], add=True)
        return 0
    lax.fori_loop(0, NCHUNK, chunk_body, 0)

    plsc.subcore_barrier()
    pltpu.sync_copy(acc_sh.at[pl.ds(s * RPT, RPT), :],
                    accp_hbm.at[c, pl.ds(s * RPT, RPT), :])
    pltpu.sync_copy(den_l, denp_hbm.at[wid])


def _sc_call(xp, asrc, adst, srcp, dstp):
    f = pl.kernel(
        _sc_body,
        out_type=(jax.ShapeDtypeStruct((2, NP, C), jnp.float32),
                  jax.ShapeDtypeStruct((NW, NP), jnp.float32)),
        mesh=plsc.VectorSubcoreMesh(core_axis_name="c", subcore_axis_name="s"),
        compiler_params=pltpu.CompilerParams(needs_layout_passes=False),
        scratch_types=[
            pltpu.VMEM((NP,), jnp.float32),
            pltpu.VMEM((NP,), jnp.float32),
            pltpu.VMEM((CHUNK,), jnp.int32),
            pltpu.VMEM((CHUNK,), jnp.int32),
            pltpu.VMEM((CHUNK,), jnp.float32),
            pltpu.VMEM((CHUNK, C), jnp.float32),
            pltpu.VMEM((NP,), jnp.float32),
            pltpu.VMEM_SHARED((NP, C), jnp.float32),
        ],
    )
    return f(xp, asrc, adst, srcp, dstp)


def _comb_body(acc_ref, den_ref, bias_ref, out_ref):
    a = acc_ref[0] + acc_ref[1]
    d = jnp.sum(den_ref[...], axis=0)
    out_ref[...] = a / (d + 1e-16)[:, None] + bias_ref[...]


def _comb_call(accp, denp, bias):
    return pl.pallas_call(
        _comb_body,
        grid=(GRID,),
        in_specs=[
            pl.BlockSpec((2, BLK, C), lambda i: (0, i, 0)),
            pl.BlockSpec((NW, BLK), lambda i: (0, i)),
            pl.BlockSpec((1, C), lambda i: (0, 0)),
        ],
        out_specs=pl.BlockSpec((BLK, C), lambda i: (i, 0)),
        out_shape=jax.ShapeDtypeStruct((NP, C), jnp.float32),
    )(accp, denp, bias)


def kernel(x, edge_index, W, att_src, att_dst, bias):
    xpad = jnp.pad(x, ((0, NP - N), (0, 0)))
    srcp = jnp.pad(edge_index[0], (0, EP - E), constant_values=NP - 1)
    dstp = jnp.pad(edge_index[1], (0, EP - E), constant_values=NP - 1)
    xp, asrc, adst = _mm_call(xpad, W, att_src.reshape(1, C),
                              att_dst.reshape(1, C))
    accp, denp = _sc_call(xp, asrc, adst, srcp, dstp)
    out = _comb_call(accp, denp, bias.reshape(1, C))
    return out[:N]


# speedup vs baseline: 1.2847x; 1.0070x over previous
import jax
import jax.numpy as jnp
from jax import lax
from jax.experimental import pallas as pl
from jax.experimental.pallas import tpu as pltpu
from jax.experimental.pallas import tpu_sc as plsc

N = 10000
NP = 10240
E = 320000
C = 128
NEG = 0.2

NW = 32
CHUNK = 128
NCHUNK = 79
EPW = NCHUNK * CHUNK
EP = NW * EPW
RPT = NP // 16
BLK = 1024
GRID = NP // BLK


def _mm_body(x_ref, w_ref, asrc_ref, adst_ref, xp_ref, as_ref, ad_ref):
    xp = jnp.dot(x_ref[...], w_ref[...], preferred_element_type=jnp.float32)
    xp_ref[...] = xp
    as_ref[...] = jnp.sum(xp * asrc_ref[...], axis=1)
    ad_ref[...] = jnp.sum(xp * adst_ref[...], axis=1)


def _mm_call(xpad, W, att_src, att_dst):
    return pl.pallas_call(
        _mm_body,
        grid=(GRID,),
        in_specs=[
            pl.BlockSpec((BLK, C), lambda i: (i, 0)),
            pl.BlockSpec((C, C), lambda i: (0, 0)),
            pl.BlockSpec((1, C), lambda i: (0, 0)),
            pl.BlockSpec((1, C), lambda i: (0, 0)),
        ],
        out_specs=[
            pl.BlockSpec((BLK, C), lambda i: (i, 0)),
            pl.BlockSpec((BLK,), lambda i: (i,)),
            pl.BlockSpec((BLK,), lambda i: (i,)),
        ],
        out_shape=[
            jax.ShapeDtypeStruct((NP, C), jnp.float32),
            jax.ShapeDtypeStruct((NP,), jnp.float32),
            jax.ShapeDtypeStruct((NP,), jnp.float32),
        ],
    )(xpad, W, att_src, att_dst)


def _sc_body(xp_hbm, asrc_hbm, adst_hbm, src_hbm, dst_hbm,
             accp_hbm, denp_hbm,
             asrc_t, adst_t, sidx, didx, wbuf, rows, den_l, acc_sh):
    c = lax.axis_index("c")
    s = lax.axis_index("s")
    wid = s * 2 + c

    pltpu.sync_copy(asrc_hbm, asrc_t)
    pltpu.sync_copy(adst_hbm, adst_t)

    zero16 = jnp.zeros((16,), jnp.float32)

    def zden(i, _):
        den_l[pl.ds(i * 16, 16)] = zero16
        return 0
    lax.fori_loop(0, NP // 16, zden, 0)

    def zrow(i, _):
        for j in range(8):
            rows[i, pl.ds(j * 16, 16)] = zero16
        return 0
    lax.fori_loop(0, CHUNK, zrow, 0)

    for r in range(RPT // CHUNK):
        pltpu.sync_copy(rows, acc_sh.at[pl.ds(s * RPT + r * CHUNK, CHUNK), :])
    plsc.subcore_barrier()

    def rmax(tbl):
        def body(i, m):
            return jnp.maximum(m, tbl[pl.ds(i * 16, 16)])
        m16 = lax.fori_loop(0, NP // 16, body,
                            jnp.full((16,), -jnp.inf, jnp.float32))
        m = m16[0]
        for i in range(1, 16):
            m = jnp.maximum(m, m16[i])
        return m
    M = rmax(asrc_t) + rmax(adst_t)

    ebase = wid * EPW

    def chunk_body(cidx, _):
        off = ebase + cidx * CHUNK
        pltpu.sync_copy(src_hbm.at[pl.ds(off, CHUNK)], sidx)
        pltpu.sync_copy(dst_hbm.at[pl.ds(off, CHUNK)], didx)
        for j in range(CHUNK // 16):
            si = sidx[pl.ds(j * 16, 16)]
            di = didx[pl.ds(j * 16, 16)]
            e = plsc.load_gather(asrc_t, [si]) + plsc.load_gather(adst_t, [di])
            e = jnp.where(e > 0, e, NEG * e)
            w = jnp.exp(e - M)
            wbuf[pl.ds(j * 16, 16)] = w
            plsc.addupdate_scatter(den_l, [di], w)
        pltpu.sync_copy(xp_hbm.at[sidx], rows)

        @plsc.parallel_loop(0, CHUNK, step=16)
        def _(r0):
            wv = wbuf[pl.ds(r0, 16)]
            for rr in range(16):
                wr = wv[rr]
                for j in range(8):
                    rows[r0 + rr, pl.ds(j * 16, 16)] = (
                        rows[r0 + rr, pl.ds(j * 16, 16)] * wr)
        pltpu.sync_copy(rows, acc_sh.at[didx], add=True)
        return 0
    lax.fori_loop(0, NCHUNK, chunk_body, 0)

    plsc.subcore_barrier()
    pltpu.sync_copy(acc_sh.at[pl.ds(s * RPT, RPT), :],
                    accp_hbm.at[c, pl.ds(s * RPT, RPT), :])
    pltpu.sync_copy(den_l, denp_hbm.at[wid])


def _sc_call(xp, asrc, adst, srcp, dstp):
    f = pl.kernel(
        _sc_body,
        out_type=(jax.ShapeDtypeStruct((2, NP, C), jnp.float32),
                  jax.ShapeDtypeStruct((NW, NP), jnp.float32)),
        mesh=plsc.VectorSubcoreMesh(core_axis_name="c", subcore_axis_name="s"),
        compiler_params=pltpu.CompilerParams(needs_layout_passes=False),
        scratch_types=[
            pltpu.VMEM((NP,), jnp.float32),
            pltpu.VMEM((NP,), jnp.float32),
            pltpu.VMEM((CHUNK,), jnp.int32),
            pltpu.VMEM((CHUNK,), jnp.int32),
            pltpu.VMEM((CHUNK,), jnp.float32),
            pltpu.VMEM((CHUNK, C), jnp.float32),
            pltpu.VMEM((NP,), jnp.float32),
            pltpu.VMEM_SHARED((NP, C), jnp.float32),
        ],
    )
    return f(xp, asrc, adst, srcp, dstp)


def _comb_body(acc_ref, den_ref, bias_ref, out_ref):
    a = acc_ref[0] + acc_ref[1]
    d = jnp.sum(den_ref[...], axis=0)
    out_ref[...] = a / (d + 1e-16)[:, None] + bias_ref[...]


def _comb_call(accp, denp, bias):
    return pl.pallas_call(
        _comb_body,
        grid=(GRID,),
        in_specs=[
            pl.BlockSpec((2, BLK, C), lambda i: (0, i, 0)),
            pl.BlockSpec((NW, BLK), lambda i: (0, i)),
            pl.BlockSpec((1, C), lambda i: (0, 0)),
        ],
        out_specs=pl.BlockSpec((BLK, C), lambda i: (i, 0)),
        out_shape=jax.ShapeDtypeStruct((NP, C), jnp.float32),
    )(accp, denp, bias)


def kernel(x, edge_index, W, att_src, att_dst, bias):
    xpad = jnp.pad(x, ((0, NP - N), (0, 0)))
    srcp = jnp.pad(edge_index[0], (0, EP - E), constant_values=NP - 1)
    dstp = jnp.pad(edge_index[1], (0, EP - E), constant_values=NP - 1)
    xp, asrc, adst = _mm_call(xpad, W, att_src.reshape(1, C),
                              att_dst.reshape(1, C))
    accp, denp = _sc_call(xp, asrc, adst, srcp, dstp)
    out = _comb_call(accp, denp, bias.reshape(1, C))
    return out[:N]
